# baseline (speedup 1.0000x reference)
import jax
import jax.numpy as jnp
from jax.experimental import pallas as pl
from jax.experimental.pallas import tpu as pltpu

EMBED = 4096
SEQLEN = 8192
BLOCK_S = 1024


def _selector_body(x_ref, w_ref, emb_ref, idx_ref, mval_ref):
    b = pl.program_id(0)
    s = pl.program_id(1)

    @pl.when(s == 0)
    def _init():
        mval_ref[0] = -jnp.inf

    x = x_ref[0]
    w = w_ref[...]
    logits = jnp.sum(x * w, axis=1, keepdims=True)

    m = jnp.max(logits)
    row_ids = jax.lax.broadcasted_iota(jnp.int32, logits.shape, 0)
    local_idx = jnp.min(jnp.where(logits == m, row_ids, BLOCK_S))

    @pl.when(m > mval_ref[0])
    def _update():
        mval_ref[0] = m
        idx_ref[b] = s * BLOCK_S + local_idx
        emb_ref[pl.ds(b, 1), :] = x_ref[0, pl.ds(local_idx, 1), :]


@jax.jit
def _run(token_embeddings, W):
    B = token_embeddings.shape[0]
    grid = (B, SEQLEN // BLOCK_S)
    emb, idx = pl.pallas_call(
        _selector_body,
        grid=grid,
        in_specs=[
            pl.BlockSpec((1, BLOCK_S, EMBED), lambda b, s: (b, s, 0)),
            pl.BlockSpec((1, EMBED), lambda b, s: (0, 0)),
        ],
        out_specs=[
            pl.BlockSpec((B, EMBED), lambda b, s: (0, 0)),
            pl.BlockSpec(memory_space=pltpu.SMEM),
        ],
        out_shape=[
            jax.ShapeDtypeStruct((B, EMBED), jnp.float32),
            jax.ShapeDtypeStruct((B,), jnp.int32),
        ],
        scratch_shapes=[pltpu.SMEM((1,), jnp.float32)],
        compiler_params=pltpu.CompilerParams(
            dimension_semantics=("arbitrary", "arbitrary"),
        ),
    )(token_embeddings, W)
    return emb, idx


def kernel(token_embeddings, W, b):
    emb, idx = _run(token_embeddings, W)
    return emb, idx
